# Initial kernel scaffold; baseline (speedup 1.0000x reference)
import jax
import jax.numpy as jnp
from jax.experimental import pallas as pl


def kernel(node_feat, edge_feat, edge_index, rand_x, rand_edge, atom_emb, bond_emb, Wn1, bn1, Wn2, bn2, We1, be1, We2, be2, W_edge, b_edge, W_nin, W_nout, W_pre, b_pre, W_post1, b_post1, W_post2, b_post2, edge_eps, node_eps, Wr1, br1, Wr2, br2):
    raise NotImplementedError("write your pallas kernel here")


# speedup vs baseline: 1.4749x; 1.4749x over previous
import functools

import jax
import jax.numpy as jnp
from jax import lax
from jax.experimental import pallas as pl
from jax.experimental.pallas import tpu as pltpu
from jax.experimental.pallas import tpu_sc as plsc

_N = 10000
_E = 160000
_H = 128
_R = 10
_L = 3
_NW = 32
_NPW = 320
_NPAD = _NW * _NPW
_C = 32
_GW = 128

_SC_MESH = plsc.VectorSubcoreMesh(core_axis_name="c", subcore_axis_name="s")


def _sc_gather(table, idx):
    b = idx.shape[0]
    d = table.shape[1]

    @functools.partial(
        pl.kernel,
        out_type=jax.ShapeDtypeStruct((b, d), table.dtype),
        mesh=_SC_MESH,
    )
    def k(tab_hbm, idx_hbm, out_hbm):
        def body(i_vmem, o_vmem):
            pltpu.sync_copy(tab_hbm.at[i_vmem.at[0]], o_vmem)

        pltpu.emit_pipeline(
            body,
            grid=(b // _GW,),
            in_specs=[pl.BlockSpec((1, _GW), lambda i: (0, i))],
            out_specs=[pl.BlockSpec((_GW, d), lambda i: (i, 0))],
            core_axis_name=("c", "s"),
            dimension_semantics=(pltpu.PARALLEL,),
        )(idx_hbm, out_hbm)

    return k(table, idx.reshape(1, b))


def _sc_segment_reduce(m, order_p, dst_sorted_p, rowptr):
    out_t = jax.ShapeDtypeStruct((_NPAD, _H), jnp.float32)

    @functools.partial(
        pl.kernel,
        out_type=(out_t, out_t, out_t),
        mesh=_SC_MESH,
        scratch_types=[
            pltpu.VMEM((_NPW, _H), jnp.float32),
            pltpu.VMEM((_NPW, _H), jnp.float32),
            pltpu.VMEM((_NPW, _H), jnp.float32),
            pltpu.VMEM((_C, _H), jnp.float32),
            pltpu.VMEM((_C,), jnp.int32),
            pltpu.VMEM((_C + 16,), jnp.int32),
            pltpu.VMEM((48,), jnp.int32),
            pltpu.SemaphoreType.DMA,
        ],
    )
    def k(m_hbm, ord_hbm, dsts_hbm, rp_hbm, sum_hbm, mx_hbm, mn_hbm,
          acc_s, acc_mx, acc_mn, mrows, idx_v, dst_sm, rp_sm, sem):
        wid = lax.axis_index("s") * 2 + lax.axis_index("c")
        n0 = wid * _NPW
        n1 = n0 + _NPW

        zero = jnp.zeros((1, 16), jnp.float32)
        neg = jnp.full((1, 16), -jnp.inf, jnp.float32)
        pos = jnp.full((1, 16), jnp.inf, jnp.float32)

        @pl.loop(0, _NPW)
        def _(r):
            for c in range(_H // 16):
                sl = (pl.ds(r, 1), pl.ds(c * 16, 16))
                acc_s[sl] = zero
                acc_mx[sl] = neg
                acc_mn[sl] = pos

        b0 = (n0 // 8) * 8
        pltpu.sync_copy(rp_hbm.at[pl.ds(b0, 8)], rp_sm.at[pl.ds(0, 8)])
        b1 = (n1 // 8) * 8
        pltpu.sync_copy(rp_hbm.at[pl.ds(b1, 8)], rp_sm.at[pl.ds(24, 8)])
        e_lo = rp_sm[pl.ds(n0 - b0, 16)][0]
        e_hi = rp_sm[pl.ds(24 + n1 - b1, 16)][0]

        c0 = (e_lo // 8) * 8
        nchunks = (e_hi - c0 + _C - 1) // _C

        def chunk(ci, carry):
            cstart = c0 + ci * _C
            pltpu.sync_copy(ord_hbm.at[pl.ds(cstart, _C)], idx_v)
            pltpu.sync_copy(dsts_hbm.at[pl.ds(cstart, _C)], dst_sm.at[pl.ds(0, _C)])
            pltpu.async_copy(m_hbm.at[idx_v], mrows, sem).wait()

            def row(r, rcarry):
                d = dst_sm[pl.ds(r, 16)][0]

                @pl.when((d >= n0) & (d < n1))
                def _():
                    rel = d - n0
                    for c in range(_H // 16):
                        ssl = pl.ds(c * 16, 16)
                        v = mrows[pl.ds(r, 1), ssl]
                        plsc.addupdate(acc_s.at[pl.ds(rel, 1), ssl], v)
                        acc_mx[pl.ds(rel, 1), ssl] = jnp.maximum(
                            acc_mx[pl.ds(rel, 1), ssl], v)
                        acc_mn[pl.ds(rel, 1), ssl] = jnp.minimum(
                            acc_mn[pl.ds(rel, 1), ssl], v)

                return rcarry

            lax.fori_loop(0, _C, row, 0)
            return carry

        lax.fori_loop(0, nchunks, chunk, 0)

        pltpu.sync_copy(acc_s, sum_hbm.at[pl.ds(n0, _NPW)])
        pltpu.sync_copy(acc_mx, mx_hbm.at[pl.ds(n0, _NPW)])
        pltpu.sync_copy(acc_mn, mn_hbm.at[pl.ds(n0, _NPW)])

    return k(m, order_p, dst_sorted_p, rowptr)


def _mm(a, b):
    return jnp.dot(a, b, preferred_element_type=jnp.float32)


def _tc_init_mlp(emb, randp, w1a, w1r, b1, w2, b2, blk):
    n = emb.shape[0]

    def body(e_ref, r_ref, w1a_ref, w1r_ref, b1_ref, w2_ref, b2_ref, o_ref):
        t = _mm(e_ref[...], w1a_ref[...]) + _mm(r_ref[...], w1r_ref[...]) + b1_ref[...]
        t = jnp.maximum(t, 0.0)
        o_ref[...] = _mm(t, w2_ref[...]) + b2_ref[...]

    row = lambda i: (i, 0)
    zero = lambda i: (0, 0)
    return pl.pallas_call(
        body,
        grid=(n // blk,),
        in_specs=[
            pl.BlockSpec((blk, _H), row), pl.BlockSpec((blk, _H), row),
            pl.BlockSpec((_H, _H), zero), pl.BlockSpec((_H, _H), zero),
            pl.BlockSpec((1, _H), zero), pl.BlockSpec((_H, _H), zero),
            pl.BlockSpec((1, _H), zero),
        ],
        out_specs=pl.BlockSpec((blk, _H), row),
        out_shape=jax.ShapeDtypeStruct((n, _H), jnp.float32),
    )(emb, randp, w1a, w1r, b1, w2, b2)


def _tc_proj(h, wa, wb, blk=512):
    n = h.shape[0]

    def body(h_ref, wa_ref, wb_ref, oa_ref, ob_ref):
        hv = h_ref[...]
        oa_ref[...] = _mm(hv, wa_ref[...])
        ob_ref[...] = _mm(hv, wb_ref[...])

    row = lambda i: (i, 0)
    zero = lambda i: (0, 0)
    t = jax.ShapeDtypeStruct((n, _H), jnp.float32)
    return pl.pallas_call(
        body,
        grid=(n // blk,),
        in_specs=[pl.BlockSpec((blk, _H), row),
                  pl.BlockSpec((_H, _H), zero), pl.BlockSpec((_H, _H), zero)],
        out_specs=[pl.BlockSpec((blk, _H), row), pl.BlockSpec((blk, _H), row)],
        out_shape=[t, t],
    )(h, wa, wb)


def _tc_edge_update(e, g1, g2, w_e, b_e, w_p, b_p, w_m, b_m, epsp1, blk=640):
    def body(eps_ref, e_ref, g1_ref, g2_ref, we_ref, be_ref, wp_ref, bp_ref,
             wm_ref, bm_ref, en_ref, m_ref):
        pre = _mm(e_ref[...], we_ref[...]) + g1_ref[...] + g2_ref[...] + be_ref[...]
        pre = jnp.maximum(pre, 0.0)
        en = eps_ref[0, 0] * e_ref[...] + _mm(pre, wp_ref[...]) + bp_ref[...]
        en_ref[...] = en
        m_ref[...] = _mm(en, wm_ref[...]) + bm_ref[...]

    row = lambda i: (i, 0)
    zero = lambda i: (0, 0)
    t = jax.ShapeDtypeStruct((_E, _H), jnp.float32)
    return pl.pallas_call(
        body,
        grid=(_E // blk,),
        in_specs=[
            pl.BlockSpec(memory_space=pltpu.SMEM),
            pl.BlockSpec((blk, _H), row), pl.BlockSpec((blk, _H), row),
            pl.BlockSpec((blk, _H), row),
            pl.BlockSpec((_H, _H), zero), pl.BlockSpec((1, _H), zero),
            pl.BlockSpec((_H, _H), zero), pl.BlockSpec((1, _H), zero),
            pl.BlockSpec((_H, _H), zero), pl.BlockSpec((1, _H), zero),
        ],
        out_specs=[pl.BlockSpec((blk, _H), row), pl.BlockSpec((blk, _H), row)],
        out_shape=[t, t],
    )(epsp1, e, g1, g2, w_e, b_e, w_p, b_p, w_m, b_m)


def _tc_node_update(h, ssum, smx, smn, cnt, wm, wx, wn, b, epsp1, blk=512):
    n = h.shape[0]

    def body(eps_ref, h_ref, ss_ref, sx_ref, sn_ref, c_ref, wm_ref, wx_ref,
             wn_ref, b_ref, o_ref):
        c = c_ref[...]
        valid = c > 0.0
        mean = ss_ref[...] / jnp.maximum(c, 1.0)
        mx = jnp.where(valid, sx_ref[...], 0.0)
        mn = jnp.where(valid, sn_ref[...], 0.0)
        o_ref[...] = (eps_ref[0, 0] * h_ref[...] + _mm(mean, wm_ref[...])
                      + _mm(mx, wx_ref[...]) + _mm(mn, wn_ref[...]) + b_ref[...])

    row = lambda i: (i, 0)
    zero = lambda i: (0, 0)
    return pl.pallas_call(
        body,
        grid=(n // blk,),
        in_specs=[
            pl.BlockSpec(memory_space=pltpu.SMEM),
            pl.BlockSpec((blk, _H), row), pl.BlockSpec((blk, _H), row),
            pl.BlockSpec((blk, _H), row), pl.BlockSpec((blk, _H), row),
            pl.BlockSpec((blk, _H), row),
            pl.BlockSpec((_H, _H), zero), pl.BlockSpec((_H, _H), zero),
            pl.BlockSpec((_H, _H), zero), pl.BlockSpec((1, _H), zero),
        ],
        out_specs=pl.BlockSpec((blk, _H), row),
        out_shape=jax.ShapeDtypeStruct((n, _H), jnp.float32),
    )(epsp1, h, ssum, smx, smn, cnt, wm, wx, wn, b)


def _tc_readout(h, wr1, br1, wr2, br2, blk=128):
    n = h.shape[0]
    steps = n // blk

    def body(h_ref, wr1_ref, br1_ref, wr2_ref, br2_ref, o_ref, acc_s, acc_m):
        i = pl.program_id(0)

        @pl.when(i == 0)
        def _():
            acc_s[...] = jnp.zeros_like(acc_s)
            acc_m[...] = jnp.full_like(acc_m, -jnp.inf)

        rows = i * blk + lax.broadcasted_iota(jnp.int32, (blk, _H), 0)
        mask = rows < _N
        hv = h_ref[...]
        hs = jnp.where(mask, hv, 0.0).reshape(blk // 8, 8, _H)
        hm = jnp.where(mask, hv, -jnp.inf).reshape(blk // 8, 8, _H)
        acc_s[...] = acc_s[...] + jnp.sum(hs, axis=0)
        acc_m[...] = jnp.maximum(acc_m[...], jnp.max(hm, axis=0))

        @pl.when(i == steps - 1)
        def _():
            s = jnp.sum(acc_s[...], axis=0, keepdims=True)
            mx = jnp.max(acc_m[...], axis=0, keepdims=True)
            ro = jnp.concatenate([s, s / float(_N), mx], axis=1)
            t = jnp.maximum(_mm(ro, wr1_ref[...]) + br1_ref[...], 0.0)
            o_ref[...] = _mm(t, wr2_ref[...]) + br2_ref[...]

    zero = lambda i: (0, 0)
    return pl.pallas_call(
        body,
        grid=(steps,),
        in_specs=[
            pl.BlockSpec((blk, _H), lambda i: (i, 0)),
            pl.BlockSpec((3 * _H, _H), zero), pl.BlockSpec((1, _H), zero),
            pl.BlockSpec((_H, _H), zero), pl.BlockSpec((1, _H), zero),
        ],
        out_specs=pl.BlockSpec((1, _H), zero),
        out_shape=jax.ShapeDtypeStruct((1, _H), jnp.float32),
        scratch_shapes=[pltpu.VMEM((8, _H), jnp.float32),
                        pltpu.VMEM((8, _H), jnp.float32)],
    )(h, wr1, br1, wr2, br2)


def kernel(node_feat, edge_feat, edge_index, rand_x, rand_edge, atom_emb,
           bond_emb, Wn1, bn1, Wn2, bn2, We1, be1, We2, be2, W_edge, b_edge,
           W_nin, W_nout, W_pre, b_pre, W_post1, b_post1, W_post2, b_post2,
           edge_eps, node_eps, Wr1, br1, Wr2, br2):
    nf = node_feat.astype(jnp.int32)
    ef = edge_feat.astype(jnp.int32)
    src = edge_index[0].astype(jnp.int32)
    dst = edge_index[1].astype(jnp.int32)

    dst_sorted, order = lax.sort_key_val(dst, jnp.arange(_E, dtype=jnp.int32))
    rowptr = jnp.searchsorted(
        dst_sorted, jnp.arange(_NPAD + 1, dtype=jnp.int32)).astype(jnp.int32)
    dst_sorted_p = jnp.concatenate(
        [dst_sorted, jnp.full((_C,), _NPAD, jnp.int32)])
    order_p = jnp.concatenate([order, jnp.zeros((_C,), jnp.int32)])
    deg = jnp.diff(rowptr).astype(jnp.float32)
    cnt_b = jnp.broadcast_to(deg[:, None], (_NPAD, _H))

    nf_p = jnp.pad(nf, (0, _NPAD - _N))
    randx_p = jnp.pad(rand_x, ((0, _NPAD - _N), (0, _H - _R)))
    rande_p = jnp.pad(rand_edge, ((0, 0), (0, _H - _R)))

    hemb = _sc_gather(atom_emb, nf_p)
    eemb = _sc_gather(bond_emb, ef)

    wn1a, wn1r = Wn1[:_H], jnp.pad(Wn1[_H:], ((0, _H - _R), (0, 0)))
    we1a, we1r = We1[:_H], jnp.pad(We1[_H:], ((0, _H - _R), (0, 0)))
    h = _tc_init_mlp(hemb, randx_p, wn1a, wn1r, bn1[None], Wn2, bn2[None], 512)
    e = _tc_init_mlp(eemb, rande_p, we1a, we1r, be1[None], We2, be2[None], 640)

    for l in range(_L):
        hwin, hwout = _tc_proj(h, W_nin[l], W_nout[l])
        g1 = _sc_gather(hwin, src)
        g2 = _sc_gather(hwout, dst)
        eps_e = (1.0 + edge_eps[l]).reshape(1, 1)
        e, m = _tc_edge_update(e, g1, g2, W_edge[l], b_edge[l][None],
                               W_pre[l], b_pre[l][None],
                               W_post1[l], b_post1[l][None], eps_e)
        ssum, smx, smn = _sc_segment_reduce(m, order_p, dst_sorted_p, rowptr)
        eps_n = (1.0 + node_eps[l]).reshape(1, 1)
        h = _tc_node_update(h, ssum, smx, smn, cnt_b,
                            W_post2[l][:_H], W_post2[l][_H:2 * _H],
                            W_post2[l][2 * _H:], b_post2[l][None], eps_n)

    return _tc_readout(h, Wr1, br1[None], Wr2, br2[None])


# speedup vs baseline: 1.6995x; 1.1523x over previous
import functools

import jax
import jax.numpy as jnp
from jax import lax
from jax.experimental import pallas as pl
from jax.experimental.pallas import tpu as pltpu
from jax.experimental.pallas import tpu_sc as plsc

_N = 10000
_E = 160000
_H = 128
_R = 10
_L = 3
_NW = 32
_NPW = 320
_NPAD = _NW * _NPW
_NSUB = 2
_NSR = _NPW // _NSUB
_CI = 512
_CM = 64
_GW = 128

_SC_MESH = plsc.VectorSubcoreMesh(core_axis_name="c", subcore_axis_name="s")


def _sc_gather(table, idx):
    b = idx.shape[0]
    d = table.shape[1]

    @functools.partial(
        pl.kernel,
        out_type=jax.ShapeDtypeStruct((b, d), table.dtype),
        mesh=_SC_MESH,
    )
    def k(tab_hbm, idx_hbm, out_hbm):
        def body(i_vmem, o_vmem):
            pltpu.sync_copy(tab_hbm.at[i_vmem.at[0]], o_vmem)

        pltpu.emit_pipeline(
            body,
            grid=(b // _GW,),
            in_specs=[pl.BlockSpec((1, _GW), lambda i: (0, i))],
            out_specs=[pl.BlockSpec((_GW, d), lambda i: (i, 0))],
            core_axis_name=("c", "s"),
            dimension_semantics=(pltpu.PARALLEL,),
        )(idx_hbm, out_hbm)

    return k(table, idx.reshape(1, b))


def _sc_segment_reduce(m, order_p, dst_sorted_p, rowptr):
    out_t = jax.ShapeDtypeStruct((_NPAD, _H), jnp.float32)
    n_mc = _CI // _CM

    @functools.partial(
        pl.kernel,
        out_type=(out_t, out_t, out_t),
        mesh=_SC_MESH,
        scratch_types=[
            pltpu.VMEM((_NSR, _H), jnp.float32),
            pltpu.VMEM((_NSR, _H), jnp.float32),
            pltpu.VMEM((_NSR, _H), jnp.float32),
            pltpu.VMEM((_CM, _H), jnp.float32),
            pltpu.VMEM((_CM, _H), jnp.float32),
            pltpu.VMEM((_CI,), jnp.int32),
            pltpu.VMEM((_CI + 16,), jnp.int32),
            pltpu.VMEM((48,), jnp.int32),
            pltpu.SemaphoreType.DMA,
            pltpu.SemaphoreType.DMA,
        ],
    )
    def k(m_hbm, ord_hbm, dsts_hbm, rp_hbm, sum_hbm, mx_hbm, mn_hbm,
          acc_s, acc_mx, acc_mn, mr0, mr1, idx_v, dst_v, rp_sm, sem0, sem1):
        wid = lax.axis_index("s") * 2 + lax.axis_index("c")
        n0 = wid * _NPW

        for s in range(_NSUB + 1):
            pltpu.sync_copy(rp_hbm.at[pl.ds(n0 + s * _NSR, 8)],
                            rp_sm.at[pl.ds(16 * s, 8)])
        bounds = [rp_sm[pl.ds(16 * s, 16)][0] for s in range(_NSUB + 1)]

        zero = jnp.zeros((1, 16), jnp.float32)
        neg = jnp.full((1, 16), -jnp.inf, jnp.float32)
        pos = jnp.full((1, 16), jnp.inf, jnp.float32)

        mbufs = (mr0, mr1)
        sems = (sem0, sem1)

        for s in range(_NSUB):
            a = n0 + s * _NSR
            b_end = a + _NSR
            e_lo, e_hi = bounds[s], bounds[s + 1]

            @pl.loop(0, _NSR)
            def _(r):
                for c in range(_H // 16):
                    sl = (pl.ds(r, 1), pl.ds(c * 16, 16))
                    acc_s[sl] = zero
                    acc_mx[sl] = neg
                    acc_mn[sl] = pos

            c0 = (e_lo // 8) * 8
            nblocks = (e_hi - c0 + _CI - 1) // _CI

            def block(bi, carry, a=a, b_end=b_end, e_hi=e_hi, c0=c0):
                bstart = c0 + bi * _CI
                pltpu.sync_copy(ord_hbm.at[pl.ds(bstart, _CI)], idx_v)
                pltpu.sync_copy(dsts_hbm.at[pl.ds(bstart, _CI)],
                                dst_v.at[pl.ds(0, _CI)])

                def valid(cj):
                    return bstart + cj * _CM < e_hi

                @pl.when(valid(0))
                def _():
                    pltpu.async_copy(m_hbm.at[idx_v.at[pl.ds(0, _CM)]],
                                     mbufs[0], sems[0])

                for cj in range(n_mc):
                    buf = mbufs[cj % 2]
                    sem = sems[cj % 2]

                    @pl.when(valid(cj))
                    def _(cj=cj, buf=buf, sem=sem):
                        if cj + 1 < n_mc:
                            @pl.when(valid(cj + 1))
                            def _():
                                pltpu.async_copy(
                                    m_hbm.at[idx_v.at[pl.ds((cj + 1) * _CM, _CM)]],
                                    mbufs[(cj + 1) % 2], sems[(cj + 1) % 2])
                        pltpu.make_async_copy(
                            m_hbm.at[idx_v.at[pl.ds(cj * _CM, _CM)]],
                            buf, sem).wait()

                        def row(j, rcarry):
                            d = dst_v[pl.ds(cj * _CM + j, 16)][0]

                            @pl.when((d >= a) & (d < b_end))
                            def _():
                                rel = d - a
                                for c in range(_H // 16):
                                    ssl = pl.ds(c * 16, 16)
                                    v = buf[pl.ds(j, 1), ssl]
                                    plsc.addupdate(
                                        acc_s.at[pl.ds(rel, 1), ssl], v)
                                    acc_mx[pl.ds(rel, 1), ssl] = jnp.maximum(
                                        acc_mx[pl.ds(rel, 1), ssl], v)
                                    acc_mn[pl.ds(rel, 1), ssl] = jnp.minimum(
                                        acc_mn[pl.ds(rel, 1), ssl], v)

                            return rcarry

                        lax.fori_loop(0, _CM, row, 0)

                return carry

            lax.fori_loop(0, nblocks, block, 0)

            pltpu.sync_copy(acc_s, sum_hbm.at[pl.ds(a, _NSR)])
            pltpu.sync_copy(acc_mx, mx_hbm.at[pl.ds(a, _NSR)])
            pltpu.sync_copy(acc_mn, mn_hbm.at[pl.ds(a, _NSR)])

    return k(m, order_p, dst_sorted_p, rowptr)


def _mm(a, b):
    return jnp.dot(a, b, preferred_element_type=jnp.float32)


def _tc_init_mlp(emb, randp, w1a, w1r, b1, w2, b2, blk):
    n = emb.shape[0]

    def body(e_ref, r_ref, w1a_ref, w1r_ref, b1_ref, w2_ref, b2_ref, o_ref):
        t = _mm(e_ref[...], w1a_ref[...]) + _mm(r_ref[...], w1r_ref[...]) + b1_ref[...]
        t = jnp.maximum(t, 0.0)
        o_ref[...] = _mm(t, w2_ref[...]) + b2_ref[...]

    row = lambda i: (i, 0)
    zero = lambda i: (0, 0)
    return pl.pallas_call(
        body,
        grid=(n // blk,),
        in_specs=[
            pl.BlockSpec((blk, _H), row), pl.BlockSpec((blk, _H), row),
            pl.BlockSpec((_H, _H), zero), pl.BlockSpec((_H, _H), zero),
            pl.BlockSpec((1, _H), zero), pl.BlockSpec((_H, _H), zero),
            pl.BlockSpec((1, _H), zero),
        ],
        out_specs=pl.BlockSpec((blk, _H), row),
        out_shape=jax.ShapeDtypeStruct((n, _H), jnp.float32),
    )(emb, randp, w1a, w1r, b1, w2, b2)


def _tc_proj(h, wa, wb, blk=512):
    n = h.shape[0]

    def body(h_ref, wa_ref, wb_ref, oa_ref, ob_ref):
        hv = h_ref[...]
        oa_ref[...] = _mm(hv, wa_ref[...])
        ob_ref[...] = _mm(hv, wb_ref[...])

    row = lambda i: (i, 0)
    zero = lambda i: (0, 0)
    t = jax.ShapeDtypeStruct((n, _H), jnp.float32)
    return pl.pallas_call(
        body,
        grid=(n // blk,),
        in_specs=[pl.BlockSpec((blk, _H), row),
                  pl.BlockSpec((_H, _H), zero), pl.BlockSpec((_H, _H), zero)],
        out_specs=[pl.BlockSpec((blk, _H), row), pl.BlockSpec((blk, _H), row)],
        out_shape=[t, t],
    )(h, wa, wb)


def _tc_edge_update(e, g1, g2, w_e, b_e, w_p, b_p, w_m, b_m, epsp1, blk=640):
    def body(eps_ref, e_ref, g1_ref, g2_ref, we_ref, be_ref, wp_ref, bp_ref,
             wm_ref, bm_ref, en_ref, m_ref):
        pre = _mm(e_ref[...], we_ref[...]) + g1_ref[...] + g2_ref[...] + be_ref[...]
        pre = jnp.maximum(pre, 0.0)
        en = eps_ref[0, 0] * e_ref[...] + _mm(pre, wp_ref[...]) + bp_ref[...]
        en_ref[...] = en
        m_ref[...] = _mm(en, wm_ref[...]) + bm_ref[...]

    row = lambda i: (i, 0)
    zero = lambda i: (0, 0)
    t = jax.ShapeDtypeStruct((_E, _H), jnp.float32)
    return pl.pallas_call(
        body,
        grid=(_E // blk,),
        in_specs=[
            pl.BlockSpec(memory_space=pltpu.SMEM),
            pl.BlockSpec((blk, _H), row), pl.BlockSpec((blk, _H), row),
            pl.BlockSpec((blk, _H), row),
            pl.BlockSpec((_H, _H), zero), pl.BlockSpec((1, _H), zero),
            pl.BlockSpec((_H, _H), zero), pl.BlockSpec((1, _H), zero),
            pl.BlockSpec((_H, _H), zero), pl.BlockSpec((1, _H), zero),
        ],
        out_specs=[pl.BlockSpec((blk, _H), row), pl.BlockSpec((blk, _H), row)],
        out_shape=[t, t],
    )(epsp1, e, g1, g2, w_e, b_e, w_p, b_p, w_m, b_m)


def _tc_node_update(h, ssum, smx, smn, cnt, wm, wx, wn, b, epsp1, blk=512):
    n = h.shape[0]

    def body(eps_ref, h_ref, ss_ref, sx_ref, sn_ref, c_ref, wm_ref, wx_ref,
             wn_ref, b_ref, o_ref):
        c = c_ref[...]
        valid = c > 0.0
        mean = ss_ref[...] / jnp.maximum(c, 1.0)
        mx = jnp.where(valid, sx_ref[...], 0.0)
        mn = jnp.where(valid, sn_ref[...], 0.0)
        o_ref[...] = (eps_ref[0, 0] * h_ref[...] + _mm(mean, wm_ref[...])
                      + _mm(mx, wx_ref[...]) + _mm(mn, wn_ref[...]) + b_ref[...])

    row = lambda i: (i, 0)
    zero = lambda i: (0, 0)
    return pl.pallas_call(
        body,
        grid=(n // blk,),
        in_specs=[
            pl.BlockSpec(memory_space=pltpu.SMEM),
            pl.BlockSpec((blk, _H), row), pl.BlockSpec((blk, _H), row),
            pl.BlockSpec((blk, _H), row), pl.BlockSpec((blk, _H), row),
            pl.BlockSpec((blk, _H), row),
            pl.BlockSpec((_H, _H), zero), pl.BlockSpec((_H, _H), zero),
            pl.BlockSpec((_H, _H), zero), pl.BlockSpec((1, _H), zero),
        ],
        out_specs=pl.BlockSpec((blk, _H), row),
        out_shape=jax.ShapeDtypeStruct((n, _H), jnp.float32),
    )(epsp1, h, ssum, smx, smn, cnt, wm, wx, wn, b)


def _tc_readout(h, wr1, br1, wr2, br2, blk=128):
    n = h.shape[0]
    steps = n // blk

    def body(h_ref, wr1_ref, br1_ref, wr2_ref, br2_ref, o_ref, acc_s, acc_m):
        i = pl.program_id(0)

        @pl.when(i == 0)
        def _():
            acc_s[...] = jnp.zeros_like(acc_s)
            acc_m[...] = jnp.full_like(acc_m, -jnp.inf)

        rows = i * blk + lax.broadcasted_iota(jnp.int32, (blk, _H), 0)
        mask = rows < _N
        hv = h_ref[...]
        hs = jnp.where(mask, hv, 0.0).reshape(blk // 8, 8, _H)
        hm = jnp.where(mask, hv, -jnp.inf).reshape(blk // 8, 8, _H)
        acc_s[...] = acc_s[...] + jnp.sum(hs, axis=0)
        acc_m[...] = jnp.maximum(acc_m[...], jnp.max(hm, axis=0))

        @pl.when(i == steps - 1)
        def _():
            s = jnp.sum(acc_s[...], axis=0, keepdims=True)
            mx = jnp.max(acc_m[...], axis=0, keepdims=True)
            ro = jnp.concatenate([s, s / float(_N), mx], axis=1)
            t = jnp.maximum(_mm(ro, wr1_ref[...]) + br1_ref[...], 0.0)
            o_ref[...] = _mm(t, wr2_ref[...]) + br2_ref[...]

    zero = lambda i: (0, 0)
    return pl.pallas_call(
        body,
        grid=(steps,),
        in_specs=[
            pl.BlockSpec((blk, _H), lambda i: (i, 0)),
            pl.BlockSpec((3 * _H, _H), zero), pl.BlockSpec((1, _H), zero),
            pl.BlockSpec((_H, _H), zero), pl.BlockSpec((1, _H), zero),
        ],
        out_specs=pl.BlockSpec((1, _H), zero),
        out_shape=jax.ShapeDtypeStruct((1, _H), jnp.float32),
        scratch_shapes=[pltpu.VMEM((8, _H), jnp.float32),
                        pltpu.VMEM((8, _H), jnp.float32)],
    )(h, wr1, br1, wr2, br2)


def kernel(node_feat, edge_feat, edge_index, rand_x, rand_edge, atom_emb,
           bond_emb, Wn1, bn1, Wn2, bn2, We1, be1, We2, be2, W_edge, b_edge,
           W_nin, W_nout, W_pre, b_pre, W_post1, b_post1, W_post2, b_post2,
           edge_eps, node_eps, Wr1, br1, Wr2, br2):
    nf = node_feat.astype(jnp.int32)
    ef = edge_feat.astype(jnp.int32)
    src = edge_index[0].astype(jnp.int32)
    dst = edge_index[1].astype(jnp.int32)

    packed = (dst.astype(jnp.uint32) << jnp.uint32(18)) | jnp.arange(
        _E, dtype=jnp.uint32)
    packed = jnp.sort(packed)
    dst_sorted = (packed >> jnp.uint32(18)).astype(jnp.int32)
    order = (packed & jnp.uint32((1 << 18) - 1)).astype(jnp.int32)
    rowptr = jnp.searchsorted(
        dst_sorted, jnp.arange(_NPAD + 1, dtype=jnp.int32)).astype(jnp.int32)
    dst_sorted_p = jnp.concatenate(
        [dst_sorted, jnp.full((_CI,), _NPAD, jnp.int32)])
    order_p = jnp.concatenate([order, jnp.zeros((_CI,), jnp.int32)])
    deg = jnp.diff(rowptr).astype(jnp.float32)
    cnt_b = jnp.broadcast_to(deg[:, None], (_NPAD, _H))

    nf_p = jnp.pad(nf, (0, _NPAD - _N))
    randx_p = jnp.pad(rand_x, ((0, _NPAD - _N), (0, _H - _R)))
    rande_p = jnp.pad(rand_edge, ((0, 0), (0, _H - _R)))

    hemb = _sc_gather(atom_emb, nf_p)
    eemb = _sc_gather(bond_emb, ef)

    wn1a, wn1r = Wn1[:_H], jnp.pad(Wn1[_H:], ((0, _H - _R), (0, 0)))
    we1a, we1r = We1[:_H], jnp.pad(We1[_H:], ((0, _H - _R), (0, 0)))
    h = _tc_init_mlp(hemb, randx_p, wn1a, wn1r, bn1[None], Wn2, bn2[None], 512)
    e = _tc_init_mlp(eemb, rande_p, we1a, we1r, be1[None], We2, be2[None], 640)

    for l in range(_L):
        hwin, hwout = _tc_proj(h, W_nin[l], W_nout[l])
        g1 = _sc_gather(hwin, src)
        g2 = _sc_gather(hwout, dst)
        eps_e = (1.0 + edge_eps[l]).reshape(1, 1)
        e, m = _tc_edge_update(e, g1, g2, W_edge[l], b_edge[l][None],
                               W_pre[l], b_pre[l][None],
                               W_post1[l], b_post1[l][None], eps_e)
        ssum, smx, smn = _sc_segment_reduce(m, order_p, dst_sorted_p, rowptr)
        eps_n = (1.0 + node_eps[l]).reshape(1, 1)
        h = _tc_node_update(h, ssum, smx, smn, cnt_b,
                            W_post2[l][:_H], W_post2[l][_H:2 * _H],
                            W_post2[l][2 * _H:], b_post2[l][None], eps_n)

    return _tc_readout(h, Wr1, br1[None], Wr2, br2[None])
